# baseline (speedup 1.0000x reference)
import functools

import jax
import jax.numpy as jnp
from jax import lax
from jax.experimental import pallas as pl
from jax.experimental.pallas import tpu as pltpu
from jax.experimental.pallas import tpu_sc as plsc

N = 10000
E = 160000
D = 256
DH = 128

NUM_TILES = 16
K = 80
NBUF = 3
EDGES_PER_TILE = E // NUM_TILES
SLABS = 5
SLAB_E = EDGES_PER_TILE // SLABS
SLAB_C = SLAB_E // K
NP = 10240
ROWS_PER_TILE = NP // NUM_TILES
RB = K
ROW_BLOCKS = ROWS_PER_TILE // RB

MM_ROWS = 2000


def _mm_body(x_ref, w_ref, o_ref):
    h = jnp.dot(x_ref[...], w_ref[...], preferred_element_type=jnp.float32)
    o_ref[0] = h[:, :DH]
    o_ref[1] = h[:, DH:]


def _matmul_split(x, W):
    n_rb = N // MM_ROWS
    return pl.pallas_call(
        _mm_body,
        grid=(n_rb,),
        in_specs=[
            pl.BlockSpec((MM_ROWS, D), lambda r: (r, 0)),
            pl.BlockSpec((D, D), lambda r: (0, 0)),
        ],
        out_specs=pl.BlockSpec((2, MM_ROWS, DH), lambda r: (0, r, 0)),
        out_shape=jax.ShapeDtypeStruct((2, N, DH), jnp.float32),
    )(x, W)


_mesh = plsc.VectorSubcoreMesh(core_axis_name="c", subcore_axis_name="s")


@functools.partial(
    pl.kernel,
    out_type=jax.ShapeDtypeStruct((N, D), jnp.float32),
    mesh=_mesh,
    scratch_types=[
        pltpu.VMEM((SLAB_E,), jnp.int32),
        pltpu.VMEM((SLAB_C, K), jnp.int32),
        pltpu.VMEM((SLAB_E,), jnp.float32),
        pltpu.VMEM((NBUF, K, DH), jnp.float32),
        pltpu.VMEM_SHARED((NP, DH), jnp.float32),
        pltpu.SemaphoreType.DMA((NBUF,)),
        pltpu.SemaphoreType.DMA((NBUF,)),
    ],
)
def _sc_aggregate(ht_hbm, src_hbm, dst_hbm, adj_hbm, out_hbm,
                  idx_v, dst_v, adj_v, rows_v, agg_sh, sem_g, sem_s):
    c = lax.axis_index("c")
    s = lax.axis_index("s")

    row_off = c * N

    def _issue_gather(k, b):
        pltpu.async_copy(ht_hbm.at[idx_v.at[pl.ds(k * K, K)]], rows_v.at[b],
                         sem_g.at[b])

    def _wait_gather(b):
        pltpu.make_async_copy(ht_hbm.at[idx_v.at[pl.ds(0, K)]], rows_v.at[b],
                              sem_g.at[b]).wait()

    def _wait_scatter(b):
        pltpu.make_async_copy(rows_v.at[b], agg_sh.at[dst_v.at[0]],
                              sem_s.at[b]).wait()

    for slab in range(SLABS):
        e0 = s * EDGES_PER_TILE + slab * SLAB_E
        pltpu.sync_copy(src_hbm.at[pl.ds(e0, SLAB_E)], idx_v)
        pltpu.sync_copy(adj_hbm.at[pl.ds(e0, SLAB_E)], adj_v)
        pltpu.sync_copy(dst_hbm.at[s, slab], dst_v)

        @plsc.parallel_loop(0, SLAB_E // 16)
        def _bias(r):
            sl = pl.ds(r * 16, 16)
            idx_v[sl] = idx_v[sl] + row_off

        _issue_gather(0, 0)
        _issue_gather(1, 1)

        if slab == 0:
            @plsc.parallel_loop(0, RB)
            def _zero_row(r):
                for j in range(DH // 16):
                    rows_v[2, r, pl.ds(j * 16, 16)] = jnp.zeros((16,),
                                                               jnp.float32)
            for b in range(ROW_BLOCKS):
                pltpu.sync_copy(rows_v.at[2],
                                agg_sh.at[pl.ds(s * ROWS_PER_TILE + b * RB,
                                                RB)])
            plsc.subcore_barrier()

        def _chunk(k, carry):
            b = k % NBUF

            @pl.when(k + 2 < SLAB_C)
            def _prep():
                nb = (k + 2) % NBUF
                @pl.when(k >= 1)
                def _free():
                    _wait_scatter(nb)
                _issue_gather(k + 2, nb)

            _wait_gather(b)

            @plsc.parallel_loop(0, K // 16, unroll=5)
            def _scale(g):
                a16 = adj_v[pl.ds(k * K + g * 16, 16)]
                for lane in range(16):
                    e = g * 16 + lane
                    a = a16[lane]
                    for j in range(DH // 16):
                        sl = pl.ds(j * 16, 16)
                        rows_v[b, e, sl] = rows_v[b, e, sl] * a

            pltpu.async_copy(rows_v.at[b], agg_sh.at[dst_v.at[k]],
                             sem_s.at[b], add=True)
            return carry
        lax.fori_loop(0, SLAB_C, _chunk, 0)

        _wait_scatter((SLAB_C - 3) % NBUF)
        _wait_scatter((SLAB_C - 2) % NBUF)
        _wait_scatter((SLAB_C - 1) % NBUF)
    plsc.subcore_barrier()

    for b in range(ROW_BLOCKS):
        rr = s * ROWS_PER_TILE + b * RB
        pltpu.sync_copy(agg_sh.at[pl.ds(rr, RB)], rows_v.at[0])

        @plsc.parallel_loop(0, RB)
        def _relu_row(r):
            for j in range(DH // 16):
                v = rows_v[0, r, pl.ds(j * 16, 16)]
                rows_v[0, r, pl.ds(j * 16, 16)] = jnp.maximum(v, 0.0)

        @pl.when(rr < N)
        def _write():
            pltpu.sync_copy(rows_v.at[0],
                            out_hbm.at[pl.ds(rr, RB), pl.ds(c * DH, DH)])


def kernel(x, edge_index, adj_values, W):
    ht = _matmul_split(x, W).reshape(2 * N, DH)
    src = edge_index[0]
    dst = edge_index[1].reshape(NUM_TILES, SLABS, SLAB_C, K)
    return _sc_aggregate(ht, src, dst, adj_values)


# speedup vs baseline: 1.0031x; 1.0031x over previous
import functools

import jax
import jax.numpy as jnp
from jax import lax
from jax.experimental import pallas as pl
from jax.experimental.pallas import tpu as pltpu
from jax.experimental.pallas import tpu_sc as plsc

N = 10000
E = 160000
D = 256
DH = 128

NUM_TILES = 16
K = 80
NBUF = 3
EDGES_PER_TILE = E // NUM_TILES
SLABS = 5
SLAB_E = EDGES_PER_TILE // SLABS
SLAB_C = SLAB_E // K
NP = 10240
ROWS_PER_TILE = NP // NUM_TILES
RB = K
ROW_BLOCKS = ROWS_PER_TILE // RB

MM_ROWS = 2000


def _mm_body(x_ref, w_ref, o_ref):
    h = jnp.dot(x_ref[...], w_ref[...], preferred_element_type=jnp.float32)
    o_ref[0] = h[:, :DH]
    o_ref[1] = h[:, DH:]


def _matmul_split(x, W):
    n_rb = N // MM_ROWS
    return pl.pallas_call(
        _mm_body,
        grid=(n_rb,),
        in_specs=[
            pl.BlockSpec((MM_ROWS, D), lambda r: (r, 0)),
            pl.BlockSpec((D, D), lambda r: (0, 0)),
        ],
        out_specs=pl.BlockSpec((2, MM_ROWS, DH), lambda r: (0, r, 0)),
        out_shape=jax.ShapeDtypeStruct((2, N, DH), jnp.float32),
    )(x, W)


_mesh = plsc.VectorSubcoreMesh(core_axis_name="c", subcore_axis_name="s")


@functools.partial(
    pl.kernel,
    out_type=jax.ShapeDtypeStruct((N, D), jnp.float32),
    mesh=_mesh,
    scratch_types=[
        pltpu.VMEM((SLAB_E,), jnp.int32),
        pltpu.VMEM((SLAB_E,), jnp.int32),
        pltpu.VMEM((NBUF, K), jnp.int32),
        pltpu.VMEM((SLAB_E,), jnp.float32),
        pltpu.VMEM((NBUF, K, DH), jnp.float32),
        pltpu.VMEM_SHARED((NP, DH), jnp.float32),
        pltpu.SemaphoreType.DMA((NBUF,)),
        pltpu.SemaphoreType.DMA((NBUF,)),
    ],
)
def _sc_aggregate(ht_hbm, src_hbm, dst_hbm, adj_hbm, out_hbm,
                  idx_v, dst_v, dstg_v, adj_v, rows_v, agg_sh, sem_g, sem_s):
    c = lax.axis_index("c")
    s = lax.axis_index("s")

    row_off = c * N

    def _issue_gather(k, b):
        pltpu.async_copy(ht_hbm.at[idx_v.at[pl.ds(k * K, K)]], rows_v.at[b],
                         sem_g.at[b])

    def _wait_gather(b):
        pltpu.make_async_copy(ht_hbm.at[idx_v.at[pl.ds(0, K)]], rows_v.at[b],
                              sem_g.at[b]).wait()

    def _wait_scatter(b):
        pltpu.make_async_copy(rows_v.at[b], agg_sh.at[dstg_v.at[0]],
                              sem_s.at[b]).wait()

    for slab in range(SLABS):
        e0 = s * EDGES_PER_TILE + slab * SLAB_E
        pltpu.sync_copy(src_hbm.at[pl.ds(e0, SLAB_E)], idx_v)
        pltpu.sync_copy(adj_hbm.at[pl.ds(e0, SLAB_E)], adj_v)
        pltpu.sync_copy(dst_hbm.at[pl.ds(e0, SLAB_E)], dst_v)

        @plsc.parallel_loop(0, SLAB_E // 16)
        def _bias(r):
            sl = pl.ds(r * 16, 16)
            idx_v[sl] = idx_v[sl] + row_off

        _issue_gather(0, 0)
        _issue_gather(1, 1)

        if slab == 0:
            @plsc.parallel_loop(0, RB)
            def _zero_row(r):
                for j in range(DH // 16):
                    rows_v[2, r, pl.ds(j * 16, 16)] = jnp.zeros((16,),
                                                               jnp.float32)
            for b in range(ROW_BLOCKS):
                pltpu.sync_copy(rows_v.at[2],
                                agg_sh.at[pl.ds(s * ROWS_PER_TILE + b * RB,
                                                RB)])
            plsc.subcore_barrier()

        def _chunk(k, carry):
            b = k % NBUF

            @pl.when(k + 2 < SLAB_C)
            def _prep():
                nb = (k + 2) % NBUF
                @pl.when(k >= 1)
                def _free():
                    _wait_scatter(nb)
                _issue_gather(k + 2, nb)

            _wait_gather(b)

            @plsc.parallel_loop(0, K // 16)
            def _stage(g):
                sl = pl.ds(g * 16, 16)
                dstg_v[b, sl] = dst_v[pl.ds(k * K + g * 16, 16)]

            @plsc.parallel_loop(0, K // 16, unroll=5)
            def _scale(g):
                a16 = adj_v[pl.ds(k * K + g * 16, 16)]
                for lane in range(16):
                    e = g * 16 + lane
                    a = a16[lane]
                    for j in range(DH // 16):
                        sl = pl.ds(j * 16, 16)
                        rows_v[b, e, sl] = rows_v[b, e, sl] * a

            pltpu.async_copy(rows_v.at[b], agg_sh.at[dstg_v.at[b]],
                             sem_s.at[b], add=True)
            return carry
        lax.fori_loop(0, SLAB_C, _chunk, 0)

        _wait_scatter((SLAB_C - 3) % NBUF)
        _wait_scatter((SLAB_C - 2) % NBUF)
        _wait_scatter((SLAB_C - 1) % NBUF)
    plsc.subcore_barrier()

    for b in range(ROW_BLOCKS):
        rr = s * ROWS_PER_TILE + b * RB
        pltpu.sync_copy(agg_sh.at[pl.ds(rr, RB)], rows_v.at[0])

        @plsc.parallel_loop(0, RB)
        def _relu_row(r):
            for j in range(DH // 16):
                v = rows_v[0, r, pl.ds(j * 16, 16)]
                rows_v[0, r, pl.ds(j * 16, 16)] = jnp.maximum(v, 0.0)

        @pl.when(rr < N)
        def _write():
            pltpu.sync_copy(rows_v.at[0],
                            out_hbm.at[pl.ds(rr, RB), pl.ds(c * DH, DH)])


def kernel(x, edge_index, adj_values, W):
    ht = _matmul_split(x, W).reshape(2 * N, DH)
    return _sc_aggregate(ht, edge_index[0], edge_index[1],
                         adj_values)


# speedup vs baseline: 1.0457x; 1.0426x over previous
import functools

import jax
import jax.numpy as jnp
from jax import lax
from jax.experimental import pallas as pl
from jax.experimental.pallas import tpu as pltpu
from jax.experimental.pallas import tpu_sc as plsc

N = 10000
E = 160000
D = 256
DH = 128

NUM_TILES = 16
K = 80
NBUF = 4
EDGES_PER_TILE = E // NUM_TILES
SLABS = 5
SLAB_E = EDGES_PER_TILE // SLABS
SLAB_C = SLAB_E // K
NP = 10240
ROWS_PER_TILE = NP // NUM_TILES
RB = K
ROW_BLOCKS = ROWS_PER_TILE // RB

MM_ROWS = 2000


def _mm_body(x_ref, w_ref, o_ref):
    h = jnp.dot(x_ref[...], w_ref[...], preferred_element_type=jnp.float32)
    o_ref[0] = h[:, :DH]
    o_ref[1] = h[:, DH:]


def _matmul_split(x, W):
    n_rb = N // MM_ROWS
    return pl.pallas_call(
        _mm_body,
        grid=(n_rb,),
        in_specs=[
            pl.BlockSpec((MM_ROWS, D), lambda r: (r, 0)),
            pl.BlockSpec((D, D), lambda r: (0, 0)),
        ],
        out_specs=pl.BlockSpec((2, MM_ROWS, DH), lambda r: (0, r, 0)),
        out_shape=jax.ShapeDtypeStruct((2, N, DH), jnp.float32),
    )(x, W)


_mesh = plsc.VectorSubcoreMesh(core_axis_name="c", subcore_axis_name="s")


@functools.partial(
    pl.kernel,
    out_type=jax.ShapeDtypeStruct((N, D), jnp.float32),
    mesh=_mesh,
    scratch_types=[
        pltpu.VMEM((SLAB_E,), jnp.int32),
        pltpu.VMEM((SLAB_E,), jnp.int32),
        pltpu.VMEM((NBUF, K), jnp.int32),
        pltpu.VMEM((SLAB_E,), jnp.float32),
        pltpu.VMEM((NBUF, K, DH), jnp.float32),
        pltpu.VMEM_SHARED((NP, DH), jnp.float32),
        pltpu.SemaphoreType.DMA((NBUF,)),
        pltpu.SemaphoreType.DMA((NBUF,)),
    ],
)
def _sc_aggregate(ht_hbm, src_hbm, dst_hbm, adj_hbm, out_hbm,
                  idx_v, dst_v, dstg_v, adj_v, rows_v, agg_sh, sem_g, sem_s):
    c = lax.axis_index("c")
    s = lax.axis_index("s")

    row_off = c * N

    def _issue_gather(k, b):
        pltpu.async_copy(ht_hbm.at[idx_v.at[pl.ds(k * K, K)]], rows_v.at[b],
                         sem_g.at[b])

    def _wait_gather(b):
        pltpu.make_async_copy(ht_hbm.at[idx_v.at[pl.ds(0, K)]], rows_v.at[b],
                              sem_g.at[b]).wait()

    def _wait_scatter(b):
        pltpu.make_async_copy(rows_v.at[b], agg_sh.at[dstg_v.at[0]],
                              sem_s.at[b]).wait()

    for slab in range(SLABS):
        e0 = s * EDGES_PER_TILE + slab * SLAB_E
        pltpu.sync_copy(src_hbm.at[pl.ds(e0, SLAB_E)], idx_v)
        pltpu.sync_copy(adj_hbm.at[pl.ds(e0, SLAB_E)], adj_v)
        pltpu.sync_copy(dst_hbm.at[pl.ds(e0, SLAB_E)], dst_v)

        @plsc.parallel_loop(0, SLAB_E // 16)
        def _bias(r):
            sl = pl.ds(r * 16, 16)
            idx_v[sl] = idx_v[sl] + row_off

        _issue_gather(0, 0)
        _issue_gather(1, 1)

        if slab == 0:
            @plsc.parallel_loop(0, RB)
            def _zero_row(r):
                for j in range(DH // 16):
                    rows_v[3, r, pl.ds(j * 16, 16)] = jnp.zeros((16,),
                                                               jnp.float32)
            for b in range(ROW_BLOCKS):
                pltpu.sync_copy(rows_v.at[3],
                                agg_sh.at[pl.ds(s * ROWS_PER_TILE + b * RB,
                                                RB)])
            plsc.subcore_barrier()

        def _chunk(k, carry):
            b = k % NBUF

            @pl.when(k + 2 < SLAB_C)
            def _prep():
                nb = (k + 2) % NBUF
                @pl.when(k >= 2)
                def _free():
                    _wait_scatter(nb)
                _issue_gather(k + 2, nb)

            _wait_gather(b)

            @plsc.parallel_loop(0, K // 16)
            def _stage(g):
                sl = pl.ds(g * 16, 16)
                dstg_v[b, sl] = dst_v[pl.ds(k * K + g * 16, 16)]

            @plsc.parallel_loop(0, K // 16, unroll=5)
            def _scale(g):
                a16 = adj_v[pl.ds(k * K + g * 16, 16)]
                for lane in range(16):
                    e = g * 16 + lane
                    a = a16[lane]
                    for j in range(DH // 16):
                        sl = pl.ds(j * 16, 16)
                        rows_v[b, e, sl] = rows_v[b, e, sl] * a

            pltpu.async_copy(rows_v.at[b], agg_sh.at[dstg_v.at[b]],
                             sem_s.at[b], add=True)
            return carry
        lax.fori_loop(0, SLAB_C, _chunk, 0)

        for d in range(NBUF):
            _wait_scatter((SLAB_C - NBUF + d) % NBUF)
    plsc.subcore_barrier()

    for b in range(ROW_BLOCKS):
        rr = s * ROWS_PER_TILE + b * RB
        pltpu.sync_copy(agg_sh.at[pl.ds(rr, RB)], rows_v.at[0])

        @plsc.parallel_loop(0, RB)
        def _relu_row(r):
            for j in range(DH // 16):
                v = rows_v[0, r, pl.ds(j * 16, 16)]
                rows_v[0, r, pl.ds(j * 16, 16)] = jnp.maximum(v, 0.0)

        @pl.when(rr < N)
        def _write():
            pltpu.sync_copy(rows_v.at[0],
                            out_hbm.at[pl.ds(rr, RB), pl.ds(c * DH, DH)])


def kernel(x, edge_index, adj_values, W):
    ht = _matmul_split(x, W).reshape(2 * N, DH)
    return _sc_aggregate(ht, edge_index[0], edge_index[1],
                         adj_values)


# speedup vs baseline: 1.0629x; 1.0164x over previous
import functools

import jax
import jax.numpy as jnp
from jax import lax
from jax.experimental import pallas as pl
from jax.experimental.pallas import tpu as pltpu
from jax.experimental.pallas import tpu_sc as plsc

N = 10000
E = 160000
D = 256
DH = 128

NUM_TILES = 16
K = 80
NBUF = 4
EDGES_PER_TILE = E // NUM_TILES
SLABS = 5
SLAB_E = EDGES_PER_TILE // SLABS
SLAB_C = SLAB_E // K
NP = 10240
ROWS_PER_TILE = NP // NUM_TILES
RB = K
ROW_BLOCKS = ROWS_PER_TILE // RB

MM_ROWS = 2000


def _mm_body(x_ref, w_ref, o_ref):
    h = jnp.dot(x_ref[...], w_ref[...], preferred_element_type=jnp.float32)
    o_ref[0] = h[:, :DH]
    o_ref[1] = h[:, DH:]


def _matmul_split(x, W):
    n_rb = N // MM_ROWS
    return pl.pallas_call(
        _mm_body,
        grid=(n_rb,),
        in_specs=[
            pl.BlockSpec((MM_ROWS, D), lambda r: (r, 0)),
            pl.BlockSpec((D, D), lambda r: (0, 0)),
        ],
        out_specs=pl.BlockSpec((2, MM_ROWS, DH), lambda r: (0, r, 0)),
        out_shape=jax.ShapeDtypeStruct((2, N, DH), jnp.float32),
    )(x, W)


_mesh = plsc.VectorSubcoreMesh(core_axis_name="c", subcore_axis_name="s")


@functools.partial(
    pl.kernel,
    out_type=jax.ShapeDtypeStruct((N, D), jnp.float32),
    mesh=_mesh,
    scratch_types=[
        pltpu.VMEM((SLAB_E,), jnp.int32),
        pltpu.VMEM((SLAB_E,), jnp.int32),
        pltpu.VMEM((NBUF, K), jnp.int32),
        pltpu.VMEM((SLAB_E,), jnp.float32),
        pltpu.VMEM((NBUF, K, DH), jnp.float32),
        pltpu.VMEM_SHARED((NP, DH), jnp.float32),
        pltpu.SemaphoreType.DMA((NBUF,)),
        pltpu.SemaphoreType.DMA((NBUF,)),
    ],
)
def _sc_aggregate(ht_hbm, src_hbm, dst_hbm, adj_hbm, out_hbm,
                  idx_v, dst_v, dstg_v, adj_v, rows_v, agg_sh, sem_g, sem_s):
    c = lax.axis_index("c")
    s = lax.axis_index("s")

    row_off = c * N

    def _issue_gather(k, b):
        pltpu.async_copy(ht_hbm.at[idx_v.at[pl.ds(k * K, K)]], rows_v.at[b],
                         sem_g.at[b])

    def _wait_gather(b):
        pltpu.make_async_copy(ht_hbm.at[idx_v.at[pl.ds(0, K)]], rows_v.at[b],
                              sem_g.at[b]).wait()

    def _wait_scatter(b):
        pltpu.make_async_copy(rows_v.at[b], agg_sh.at[dstg_v.at[0]],
                              sem_s.at[b]).wait()

    def _refill(slab):
        e0 = s * EDGES_PER_TILE + slab * SLAB_E
        pltpu.sync_copy(src_hbm.at[pl.ds(e0, SLAB_E)], idx_v)
        pltpu.sync_copy(adj_hbm.at[pl.ds(e0, SLAB_E)], adj_v)
        pltpu.sync_copy(dst_hbm.at[pl.ds(e0, SLAB_E)], dst_v)

        @plsc.parallel_loop(0, SLAB_E // 16)
        def _bias(r):
            sl = pl.ds(r * 16, 16)
            idx_v[sl] = idx_v[sl] + row_off

    _refill(0)
    for slab in range(SLABS):
        _issue_gather(0, 0)
        _issue_gather(1, 1)

        if slab == 0:
            @plsc.parallel_loop(0, RB)
            def _zero_row(r):
                for j in range(DH // 16):
                    rows_v[3, r, pl.ds(j * 16, 16)] = jnp.zeros((16,),
                                                               jnp.float32)
            for b in range(ROW_BLOCKS):
                pltpu.sync_copy(rows_v.at[3],
                                agg_sh.at[pl.ds(s * ROWS_PER_TILE + b * RB,
                                                RB)])
            plsc.subcore_barrier()

        def _chunk(k, carry):
            b = k % NBUF

            @pl.when(k + 2 < SLAB_C)
            def _prep():
                nb = (k + 2) % NBUF
                @pl.when(k >= 2)
                def _free():
                    _wait_scatter(nb)
                _issue_gather(k + 2, nb)

            _wait_gather(b)

            @plsc.parallel_loop(0, K // 16)
            def _stage(g):
                sl = pl.ds(g * 16, 16)
                dstg_v[b, sl] = dst_v[pl.ds(k * K + g * 16, 16)]

            @plsc.parallel_loop(0, K // 16, unroll=5)
            def _scale(g):
                a16 = adj_v[pl.ds(k * K + g * 16, 16)]
                for lane in range(16):
                    e = g * 16 + lane
                    a = a16[lane]
                    for j in range(DH // 16):
                        sl = pl.ds(j * 16, 16)
                        rows_v[b, e, sl] = rows_v[b, e, sl] * a

            pltpu.async_copy(rows_v.at[b], agg_sh.at[dstg_v.at[b]],
                             sem_s.at[b], add=True)
            return carry
        lax.fori_loop(0, SLAB_C, _chunk, 0)

        if slab + 1 < SLABS:
            _refill(slab + 1)
        for d in range(NBUF):
            _wait_scatter((SLAB_C - NBUF + d) % NBUF)
    plsc.subcore_barrier()

    for b in range(ROW_BLOCKS):
        rr = s * ROWS_PER_TILE + b * RB
        pltpu.sync_copy(agg_sh.at[pl.ds(rr, RB)], rows_v.at[0])

        @plsc.parallel_loop(0, RB)
        def _relu_row(r):
            for j in range(DH // 16):
                v = rows_v[0, r, pl.ds(j * 16, 16)]
                rows_v[0, r, pl.ds(j * 16, 16)] = jnp.maximum(v, 0.0)

        @pl.when(rr < N)
        def _write():
            pltpu.sync_copy(rows_v.at[0],
                            out_hbm.at[pl.ds(rr, RB), pl.ds(c * DH, DH)])


def kernel(x, edge_index, adj_values, W):
    ht = _matmul_split(x, W).reshape(2 * N, DH)
    return _sc_aggregate(ht, edge_index[0], edge_index[1],
                         adj_values)


# speedup vs baseline: 1.0770x; 1.0133x over previous
import functools

import jax
import jax.numpy as jnp
from jax import lax
from jax.experimental import pallas as pl
from jax.experimental.pallas import tpu as pltpu
from jax.experimental.pallas import tpu_sc as plsc

N = 10000
E = 160000
D = 256
DH = 128

NUM_TILES = 16
K = 80
NBUF = 4
EDGES_PER_TILE = E // NUM_TILES
SLABS = 5
SLAB_E = EDGES_PER_TILE // SLABS
SLAB_C = SLAB_E // K
NP = 10240
ROWS_PER_TILE = NP // NUM_TILES
RB = K
ROW_BLOCKS = ROWS_PER_TILE // RB

MM_ROWS = 2000


def _mm_body(x_ref, w_ref, o_ref):
    h = jnp.dot(x_ref[...], w_ref[...], preferred_element_type=jnp.float32)
    o_ref[0] = h[:, :DH]
    o_ref[1] = h[:, DH:]


def _matmul_split(x, W):
    n_rb = N // MM_ROWS
    return pl.pallas_call(
        _mm_body,
        grid=(n_rb,),
        in_specs=[
            pl.BlockSpec((MM_ROWS, D), lambda r: (r, 0)),
            pl.BlockSpec((D, D), lambda r: (0, 0)),
        ],
        out_specs=pl.BlockSpec((2, MM_ROWS, DH), lambda r: (0, r, 0)),
        out_shape=jax.ShapeDtypeStruct((2, N, DH), jnp.float32),
    )(x, W)


_mesh = plsc.VectorSubcoreMesh(core_axis_name="c", subcore_axis_name="s")


@functools.partial(
    pl.kernel,
    out_type=jax.ShapeDtypeStruct((N, D), jnp.float32),
    mesh=_mesh,
    scratch_types=[
        pltpu.VMEM((SLAB_E,), jnp.int32),
        pltpu.VMEM((SLAB_E,), jnp.int32),
        pltpu.VMEM((NBUF, K), jnp.int32),
        pltpu.VMEM((SLAB_E,), jnp.float32),
        pltpu.VMEM((NBUF, K, DH), jnp.float32),
        pltpu.VMEM_SHARED((NP, DH), jnp.float32),
        pltpu.SemaphoreType.DMA((NBUF,)),
        pltpu.SemaphoreType.DMA((NBUF,)),
    ],
)
def _sc_aggregate(ht_hbm, src_hbm, dst_hbm, adj_hbm, out_hbm,
                  idx_v, dst_v, dstg_v, adj_v, rows_v, agg_sh, sem_g, sem_s):
    c = lax.axis_index("c")
    s = lax.axis_index("s")

    row_off = c * N

    def _issue_gather(k, b):
        pltpu.async_copy(ht_hbm.at[idx_v.at[pl.ds(k * K, K)]], rows_v.at[b],
                         sem_g.at[b])

    def _wait_gather(b):
        pltpu.make_async_copy(ht_hbm.at[idx_v.at[pl.ds(0, K)]], rows_v.at[b],
                              sem_g.at[b]).wait()

    def _wait_scatter(b):
        pltpu.make_async_copy(rows_v.at[b], agg_sh.at[dstg_v.at[0]],
                              sem_s.at[b]).wait()

    def _refill(slab):
        e0 = s * EDGES_PER_TILE + slab * SLAB_E
        pltpu.sync_copy(src_hbm.at[pl.ds(e0, SLAB_E)], idx_v)
        pltpu.sync_copy(adj_hbm.at[pl.ds(e0, SLAB_E)], adj_v)
        pltpu.sync_copy(dst_hbm.at[pl.ds(e0, SLAB_E)], dst_v)

        @plsc.parallel_loop(0, SLAB_E // 16)
        def _bias(r):
            sl = pl.ds(r * 16, 16)
            idx_v[sl] = idx_v[sl] + row_off

    _refill(0)
    for slab in range(SLABS):
        _issue_gather(0, 0)
        _issue_gather(1, 1)

        if slab == 0:
            @plsc.parallel_loop(0, RB)
            def _zero_row(r):
                for j in range(DH // 16):
                    rows_v[3, r, pl.ds(j * 16, 16)] = jnp.zeros((16,),
                                                               jnp.float32)
            for b in range(ROW_BLOCKS):
                pltpu.sync_copy(rows_v.at[3],
                                agg_sh.at[pl.ds(s * ROWS_PER_TILE + b * RB,
                                                RB)])
            plsc.subcore_barrier()

        def _chunk(k, carry):
            b = k % NBUF

            @pl.when(k + 2 < SLAB_C)
            def _prep():
                nb = (k + 2) % NBUF
                @pl.when(k >= 2)
                def _free():
                    _wait_scatter(nb)
                _issue_gather(k + 2, nb)

            _wait_gather(b)

            @plsc.parallel_loop(0, K // 16)
            def _stage(g):
                sl = pl.ds(g * 16, 16)
                dstg_v[b, sl] = dst_v[pl.ds(k * K + g * 16, 16)]

            @plsc.parallel_loop(0, K // 16, unroll=5)
            def _scale(g):
                a16 = adj_v[pl.ds(k * K + g * 16, 16)]
                for lane in range(16):
                    e = g * 16 + lane
                    a = a16[lane]
                    for j in range(DH // 16):
                        sl = pl.ds(j * 16, 16)
                        rows_v[b, e, sl] = rows_v[b, e, sl] * a

            pltpu.async_copy(rows_v.at[b], agg_sh.at[dstg_v.at[b]],
                             sem_s.at[b], add=True)
            return carry
        lax.fori_loop(0, SLAB_C, _chunk, 0)

        if slab + 1 < SLABS:
            _refill(slab + 1)
        for d in range(NBUF):
            _wait_scatter((SLAB_C - NBUF + d) % NBUF)
    plsc.subcore_barrier()

    def _wait_write(b):
        bb = b % 2
        rrp = s * ROWS_PER_TILE + b * RB
        @pl.when(rrp < N)
        def _w():
            pltpu.make_async_copy(
                rows_v.at[bb],
                out_hbm.at[pl.ds(rrp, RB), pl.ds(c * DH, DH)],
                sem_g.at[bb]).wait()

    for b in range(ROW_BLOCKS):
        bb = b % 2
        if b >= 2:
            _wait_write(b - 2)
        rr = s * ROWS_PER_TILE + b * RB
        pltpu.sync_copy(agg_sh.at[pl.ds(rr, RB)], rows_v.at[bb])

        @plsc.parallel_loop(0, RB)
        def _relu_row(r):
            for j in range(DH // 16):
                v = rows_v[bb, r, pl.ds(j * 16, 16)]
                rows_v[bb, r, pl.ds(j * 16, 16)] = jnp.maximum(v, 0.0)

        @pl.when(rr < N)
        def _write():
            pltpu.async_copy(rows_v.at[bb],
                             out_hbm.at[pl.ds(rr, RB), pl.ds(c * DH, DH)],
                             sem_g.at[bb])
    _wait_write(ROW_BLOCKS - 2)
    _wait_write(ROW_BLOCKS - 1)


def kernel(x, edge_index, adj_values, W):
    ht = _matmul_split(x, W).reshape(2 * N, DH)
    return _sc_aggregate(ht, edge_index[0], edge_index[1],
                         adj_values)
